# Initial kernel scaffold; baseline (speedup 1.0000x reference)
import jax
import jax.numpy as jnp
from jax.experimental import pallas as pl


def kernel(x, gate_w, gate_proj_w, up_proj_w, down_proj_w):
    raise NotImplementedError("write your pallas kernel here")


# speedup vs baseline: 1.0745x; 1.0745x over previous
import functools

import jax
import jax.numpy as jnp
from jax.experimental import pallas as pl
from jax.experimental.pallas import tpu as pltpu

H = 2048
I = 2048
E = 8
K = 2

TM = 1024
I2 = 2
TI = I // I2


def _ffn_body(x_ref, wg_ref, wu_ref, wd_ref, wfull_ref, out_ref):
    e = pl.program_id(1)
    i2 = pl.program_id(2)
    xb = x_ref[...]
    g = jnp.dot(xb, wg_ref[0].T, preferred_element_type=jnp.float32)
    u = jnp.dot(xb, wu_ref[0].T, preferred_element_type=jnp.float32)
    h = (g * jax.nn.sigmoid(g) * u).astype(jnp.bfloat16)
    y = jnp.dot(h, wd_ref[0].T, preferred_element_type=jnp.float32)
    onehot = (jax.lax.broadcasted_iota(jnp.int32, (1, E), 1) == e).astype(jnp.float32)
    w_col = jnp.sum(wfull_ref[...] * onehot, axis=1, keepdims=True)
    contrib = w_col * y

    @pl.when(jnp.logical_and(e == 0, i2 == 0))
    def _init():
        out_ref[...] = contrib

    @pl.when(jnp.logical_not(jnp.logical_and(e == 0, i2 == 0)))
    def _acc():
        out_ref[...] += contrib


def _dense_moe(xf_bf, wg_bf, wu_bf, wd_bf, w_full):
    T = xf_bf.shape[0]
    grid = (T // TM, E, I2)
    return pl.pallas_call(
        _ffn_body,
        grid=grid,
        in_specs=[
            pl.BlockSpec((TM, H), lambda m, e, i2: (m, 0)),
            pl.BlockSpec((1, TI, H), lambda m, e, i2: (e, i2, 0)),
            pl.BlockSpec((1, TI, H), lambda m, e, i2: (e, i2, 0)),
            pl.BlockSpec((1, H, TI), lambda m, e, i2: (e, 0, i2)),
            pl.BlockSpec((TM, E), lambda m, e, i2: (m, 0)),
        ],
        out_specs=pl.BlockSpec((TM, H), lambda m, e, i2: (m, 0)),
        out_shape=jax.ShapeDtypeStruct((T, H), jnp.float32),
        compiler_params=pltpu.CompilerParams(
            dimension_semantics=("arbitrary", "arbitrary", "arbitrary"),
        ),
    )(xf_bf, wg_bf, wu_bf, wd_bf, w_full)


def kernel(x, gate_w, gate_proj_w, up_proj_w, down_proj_w):
    shape = x.shape
    xf = x.reshape(-1, shape[-1])
    T = xf.shape[0]
    logits = xf @ gate_w.T
    probs = jax.nn.softmax(logits.astype(jnp.float32), axis=-1)
    top_w, top_i = jax.lax.top_k(probs, K)
    top_w = (top_w / jnp.sum(top_w, axis=-1, keepdims=True)).astype(x.dtype)
    tok = jnp.arange(T)
    w_full = jnp.zeros((T, E), dtype=x.dtype).at[tok[:, None], top_i].add(top_w)

    out = _dense_moe(
        xf.astype(jnp.bfloat16),
        gate_proj_w.astype(jnp.bfloat16),
        up_proj_w.astype(jnp.bfloat16),
        down_proj_w.astype(jnp.bfloat16),
        w_full,
    )
    return out.reshape(shape)
